# Initial kernel scaffold; baseline (speedup 1.0000x reference)
import jax
import jax.numpy as jnp
from jax.experimental import pallas as pl


def kernel(x, edge_index, edge_type, relvectors, W_ih, W_hh, b_ih, b_hh):
    raise NotImplementedError("write your pallas kernel here")


# speedup vs baseline: 2.8294x; 2.8294x over previous
import functools

import jax
import jax.numpy as jnp
from jax import lax
from jax.experimental import pallas as pl
from jax.experimental.pallas import tpu as pltpu
from jax.experimental.pallas import tpu_sc as plsc

N = 10000
E = 320000
H = 128
R = 16

NC = 2
NS = 16
NW = NC * NS

EPW = E // NW
B = 80
NB = EPW // B
NPAD = 10240
RPT = NPAD // NS
ZR = 128


def _sc_segment_sum(x, src, dst, et, relvec):
    mesh = plsc.VectorSubcoreMesh(core_axis_name="c", subcore_axis_name="s")

    @functools.partial(
        pl.kernel,
        mesh=mesh,
        out_type=jax.ShapeDtypeStruct((NC, NPAD, H), jnp.float32),
        scratch_types=[
            pltpu.VMEM((B,), jnp.int32),
            pltpu.VMEM((B,), jnp.int32),
            pltpu.VMEM((B,), jnp.int32),
            pltpu.VMEM((B, H), jnp.float32),
            pltpu.VMEM((B, H), jnp.float32),
            pltpu.VMEM((ZR, H), jnp.float32),
            pltpu.VMEM_SHARED((NPAD, H), jnp.float32),
            pltpu.SemaphoreType.DMA,
            pltpu.SemaphoreType.DMA,
        ],
    )
    def sc_kern(x_hbm, src_hbm, dst_hbm, et_hbm, rel_hbm, out_hbm,
                srcv, dstv, etv, rows, relrows, zb, acc, sem_i, sem_g):
        cid = lax.axis_index("c")
        sid = lax.axis_index("s")
        wid = cid * NS + sid

        def zrow(i, _):
            for j in range(H // 16):
                zb[i, pl.ds(j * 16, 16)] = jnp.zeros((16,), jnp.float32)
            return 0
        lax.fori_loop(0, ZR, zrow, 0)

        abase = sid * RPT
        def zacc(k, _):
            pltpu.sync_copy(zb, acc.at[pl.ds(abase + k * ZR, ZR)])
            return 0
        lax.fori_loop(0, RPT // ZR, zacc, 0)
        plsc.subcore_barrier()

        ebase = wid * EPW
        def body(b, _):
            base = pl.multiple_of(ebase + b * B, 8)
            c1 = pltpu.async_copy(src_hbm.at[pl.ds(base, B)], srcv, sem_i)
            c2 = pltpu.async_copy(dst_hbm.at[pl.ds(base, B)], dstv, sem_i)
            c3 = pltpu.async_copy(et_hbm.at[pl.ds(base, B)], etv, sem_i)
            c1.wait(); c2.wait(); c3.wait()
            pltpu.async_copy(x_hbm.at[srcv], rows, sem_g).wait()
            pltpu.async_copy(rel_hbm.at[etv], relrows, sem_g).wait()
            pltpu.sync_copy(rows, acc.at[dstv], add=True)
            pltpu.sync_copy(relrows, acc.at[dstv], add=True)
            return 0
        lax.fori_loop(0, NB, body, 0)

        plsc.subcore_barrier()

        def ocp(k, _):
            r0 = pl.multiple_of(abase + k * ZR, 8)
            pltpu.sync_copy(acc.at[pl.ds(r0, ZR)], zb)
            pltpu.sync_copy(zb, out_hbm.at[cid, pl.ds(r0, ZR)])
            return 0
        lax.fori_loop(0, RPT // ZR, ocp, 0)

    return sc_kern(x, src, dst, et, relvec)


def _gru_body(x_ref, p_ref, wiT_ref, whT_ref, bi_ref, bh_ref, o_ref):
    red = p_ref[0] + p_ref[1]
    gi = jnp.dot(red, wiT_ref[:], preferred_element_type=jnp.float32) + bi_ref[:]
    gh = jnp.dot(x_ref[:], whT_ref[:], preferred_element_type=jnp.float32) + bh_ref[:]
    r = jax.nn.sigmoid(gi[:, :H] + gh[:, :H])
    z = jax.nn.sigmoid(gi[:, H:2 * H] + gh[:, H:2 * H])
    n = jnp.tanh(gi[:, 2 * H:] + r * gh[:, 2 * H:])
    o_ref[:] = (1.0 - z) * n + z * x_ref[:]


def _gru(x, parts, W_ih, W_hh, b_ih, b_hh):
    BN = 1000
    grid = (N // BN,)
    return pl.pallas_call(
        _gru_body,
        grid=grid,
        in_specs=[
            pl.BlockSpec((BN, H), lambda i: (i, 0)),
            pl.BlockSpec((NC, BN, H), lambda i: (0, i, 0)),
            pl.BlockSpec((H, 3 * H), lambda i: (0, 0)),
            pl.BlockSpec((H, 3 * H), lambda i: (0, 0)),
            pl.BlockSpec((1, 3 * H), lambda i: (0, 0)),
            pl.BlockSpec((1, 3 * H), lambda i: (0, 0)),
        ],
        out_specs=pl.BlockSpec((BN, H), lambda i: (i, 0)),
        out_shape=jax.ShapeDtypeStruct((N, H), jnp.float32),
    )(x, parts, W_ih.T, W_hh.T, b_ih.reshape(1, -1), b_hh.reshape(1, -1))


def kernel(x, edge_index, edge_type, relvectors, W_ih, W_hh, b_ih, b_hh):
    src = edge_index[0]
    dst = edge_index[1]
    parts = _sc_segment_sum(x, src, dst, edge_type, relvectors)
    return _gru(x, parts, W_ih, W_hh, b_ih, b_hh)


# speedup vs baseline: 2.8398x; 1.0037x over previous
import functools

import jax
import jax.numpy as jnp
from jax import lax
from jax.experimental import pallas as pl
from jax.experimental.pallas import tpu as pltpu
from jax.experimental.pallas import tpu_sc as plsc

N = 10000
E = 320000
H = 128
R = 16

NC = 2
NS = 16
NW = NC * NS

EPW = E // NW
B = 40
NB = EPW // B
NPAD = 10240
RPT = NPAD // NS
ZR = 64


def _sc_segment_sum(x, src3, dst4, et3, relvec):
    mesh = plsc.VectorSubcoreMesh(core_axis_name="c", subcore_axis_name="s")

    @functools.partial(
        pl.kernel,
        mesh=mesh,
        out_type=jax.ShapeDtypeStruct((NC, NPAD, H), jnp.float32),
        scratch_types=[
            pltpu.VMEM((B,), jnp.int32),
            pltpu.VMEM((B,), jnp.int32),
            pltpu.VMEM((B,), jnp.int32),
            pltpu.VMEM((B,), jnp.int32),
            pltpu.VMEM((1, B), jnp.int32),
            pltpu.VMEM((1, B), jnp.int32),
            pltpu.VMEM((B, H), jnp.float32),
            pltpu.VMEM((B, H), jnp.float32),
            pltpu.VMEM((B, H), jnp.float32),
            pltpu.VMEM((B, H), jnp.float32),
            pltpu.VMEM((ZR, H), jnp.float32),
            pltpu.VMEM_SHARED((NPAD, H), jnp.float32),
            pltpu.SemaphoreType.DMA,
            pltpu.SemaphoreType.DMA,
            pltpu.SemaphoreType.DMA,
            pltpu.SemaphoreType.DMA,
            pltpu.SemaphoreType.DMA,
            pltpu.SemaphoreType.DMA,
            pltpu.SemaphoreType.DMA,
            pltpu.SemaphoreType.DMA,
        ],
    )
    def sc_kern(x_hbm, src_hbm, dst_hbm, et_hbm, rel_hbm, out_hbm,
                srcb0, srcb1, etb0, etb1, dstb0, dstb1,
                rows0, rows1, rel0, rel1, zb, acc,
                si0, si1, sd0, sd1, sg0, sg1, ss0, ss1):
        cid = lax.axis_index("c")
        sid = lax.axis_index("s")
        wid = cid * NS + sid

        srcb = (srcb0, srcb1)
        etb = (etb0, etb1)
        dstb = (dstb0, dstb1)
        rows = (rows0, rows1)
        relr = (rel0, rel1)
        si = (si0, si1)
        sd = (sd0, sd1)
        sg = (sg0, sg1)
        ss = (ss0, ss1)

        def zrow(i, _):
            for j in range(H // 16):
                zb[i, pl.ds(j * 16, 16)] = jnp.zeros((16,), jnp.float32)
            return 0
        lax.fori_loop(0, ZR, zrow, 0)

        abase = sid * RPT
        def zacc(k, _):
            pltpu.sync_copy(zb, acc.at[pl.ds(abase + k * ZR, ZR)])
            return 0
        lax.fori_loop(0, RPT // ZR, zacc, 0)
        plsc.subcore_barrier()

        def fire_idx(t, p):
            pltpu.async_copy(src_hbm.at[wid, t], srcb[p], si[p])
            pltpu.async_copy(et_hbm.at[wid, t], etb[p], si[p])

        def wait_idx(p):
            pltpu.make_async_copy(src_hbm.at[wid, 0], srcb[p], si[p]).wait()
            pltpu.make_async_copy(et_hbm.at[wid, 0], etb[p], si[p]).wait()

        def fire_dst(t, p):
            pltpu.async_copy(dst_hbm.at[wid, t], dstb[p], sd[p])

        def wait_dst(p):
            pltpu.make_async_copy(dst_hbm.at[wid, 0], dstb[p], sd[p]).wait()

        def fire_gather(p):
            pltpu.async_copy(x_hbm.at[srcb[p]], rows[p], sg[p])
            pltpu.async_copy(rel_hbm.at[etb[p]], relr[p], sg[p])

        def wait_gather(p):
            pltpu.make_async_copy(x_hbm.at[srcb[p]], rows[p], sg[p]).wait()
            pltpu.make_async_copy(rel_hbm.at[etb[p]], relr[p], sg[p]).wait()

        def fire_scatter(p):
            idx = dstb[p].at[0]
            pltpu.async_copy(rows[p], acc.at[idx], ss[p], add=True)
            pltpu.async_copy(relr[p], acc.at[idx], ss[p], add=True)

        def wait_scatter(p):
            idx = dstb[p].at[0]
            pltpu.make_async_copy(rows[p], acc.at[idx], ss[p]).wait()
            pltpu.make_async_copy(relr[p], acc.at[idx], ss[p]).wait()

        fire_idx(0, 0)
        fire_dst(0, 0)
        fire_idx(1, 1)
        fire_dst(1, 1)
        wait_idx(0)
        fire_gather(0)
        wait_idx(1)
        fire_gather(1)

        def body(g, _):
            t0 = 2 * g - 2
            wait_gather(0)
            fire_idx(t0 + 2, 0)
            wait_dst(0)
            fire_scatter(0)
            wait_gather(1)
            fire_idx(t0 + 3, 1)
            wait_dst(1)
            fire_scatter(1)
            wait_scatter(0)
            fire_dst(t0 + 2, 0)
            wait_idx(0)
            fire_gather(0)
            wait_scatter(1)
            fire_dst(t0 + 3, 1)
            wait_idx(1)
            fire_gather(1)
            return 0
        lax.fori_loop(1, NB // 2, body, 0)

        wait_gather(0)
        wait_dst(0)
        fire_scatter(0)
        wait_gather(1)
        wait_dst(1)
        fire_scatter(1)
        wait_scatter(0)
        wait_scatter(1)

        plsc.subcore_barrier()

        def ocp(k, _):
            r0 = pl.multiple_of(abase + k * ZR, 8)
            pltpu.sync_copy(acc.at[pl.ds(r0, ZR)], zb)
            pltpu.sync_copy(zb, out_hbm.at[cid, pl.ds(r0, ZR)])
            return 0
        lax.fori_loop(0, RPT // ZR, ocp, 0)

    return sc_kern(x, src3, dst4, et3, relvec)


def _gru_body(x_ref, p_ref, wiT_ref, whT_ref, bi_ref, bh_ref, o_ref):
    red = p_ref[0] + p_ref[1]
    gi = jnp.dot(red, wiT_ref[:], preferred_element_type=jnp.float32) + bi_ref[:]
    gh = jnp.dot(x_ref[:], whT_ref[:], preferred_element_type=jnp.float32) + bh_ref[:]
    r = jax.nn.sigmoid(gi[:, :H] + gh[:, :H])
    z = jax.nn.sigmoid(gi[:, H:2 * H] + gh[:, H:2 * H])
    n = jnp.tanh(gi[:, 2 * H:] + r * gh[:, 2 * H:])
    o_ref[:] = (1.0 - z) * n + z * x_ref[:]


def _gru(x, parts, W_ih, W_hh, b_ih, b_hh):
    BN = 1000
    grid = (N // BN,)
    return pl.pallas_call(
        _gru_body,
        grid=grid,
        in_specs=[
            pl.BlockSpec((BN, H), lambda i: (i, 0)),
            pl.BlockSpec((NC, BN, H), lambda i: (0, i, 0)),
            pl.BlockSpec((H, 3 * H), lambda i: (0, 0)),
            pl.BlockSpec((H, 3 * H), lambda i: (0, 0)),
            pl.BlockSpec((1, 3 * H), lambda i: (0, 0)),
            pl.BlockSpec((1, 3 * H), lambda i: (0, 0)),
        ],
        out_specs=pl.BlockSpec((BN, H), lambda i: (i, 0)),
        out_shape=jax.ShapeDtypeStruct((N, H), jnp.float32),
    )(x, parts, W_ih.T, W_hh.T, b_ih.reshape(1, -1), b_hh.reshape(1, -1))


def kernel(x, edge_index, edge_type, relvectors, W_ih, W_hh, b_ih, b_hh):
    src3 = edge_index[0].reshape(NW, NB, B)
    dst4 = edge_index[1].reshape(NW, NB, 1, B)
    et3 = edge_type.reshape(NW, NB, B)
    parts = _sc_segment_sum(x, src3, dst4, et3, relvectors)
    return _gru(x, parts, W_ih, W_hh, b_ih, b_hh)


# speedup vs baseline: 9.3913x; 3.3070x over previous
import functools

import jax
import jax.numpy as jnp
from jax import lax
from jax.experimental import pallas as pl
from jax.experimental.pallas import tpu as pltpu
from jax.experimental.pallas import tpu_sc as plsc

N = 10000
E = 320000
H = 128
R = 16

NC = 2
NS = 16
NW = NC * NS

EPW = E // NW
B = 40
NB = EPW // B
NPAD = 10240
RPT = NPAD // NS
ZR = 64


def _expand_body(x_ref, rel_ref, o_ref):
    bn = x_ref.shape[0]
    msg = x_ref[:][:, None, :] + rel_ref[:][None, :, :]
    o_ref[:] = msg.reshape(bn * R, H)


def _expand(x, relvec):
    BNE = 1000
    return pl.pallas_call(
        _expand_body,
        grid=(N // BNE,),
        in_specs=[
            pl.BlockSpec((BNE, H), lambda i: (i, 0)),
            pl.BlockSpec((R, H), lambda i: (0, 0)),
        ],
        out_specs=pl.BlockSpec((BNE * R, H), lambda i: (i, 0)),
        out_shape=jax.ShapeDtypeStruct((N * R, H), jnp.float32),
    )(x, relvec)


def _cidx_body(s_ref, t_ref, o_ref):
    o_ref[:] = s_ref[:] * R + t_ref[:]


def _cidx(src2, et2):
    ROWS = E // 128
    BR = ROWS
    return pl.pallas_call(
        _cidx_body,
        grid=(ROWS // BR,),
        in_specs=[
            pl.BlockSpec((BR, 128), lambda i: (i, 0)),
            pl.BlockSpec((BR, 128), lambda i: (i, 0)),
        ],
        out_specs=pl.BlockSpec((BR, 128), lambda i: (i, 0)),
        out_shape=jax.ShapeDtypeStruct((ROWS, 128), jnp.int32),
    )(src2, et2)


def _sc_segment_sum(xrel, cidx3, dst4):
    mesh = plsc.VectorSubcoreMesh(core_axis_name="c", subcore_axis_name="s")

    @functools.partial(
        pl.kernel,
        mesh=mesh,
        out_type=jax.ShapeDtypeStruct((NC, NPAD, H), jnp.float32),
        scratch_types=[
            pltpu.VMEM((B,), jnp.int32),
            pltpu.VMEM((B,), jnp.int32),
            pltpu.VMEM((1, B), jnp.int32),
            pltpu.VMEM((1, B), jnp.int32),
            pltpu.VMEM((B, H), jnp.float32),
            pltpu.VMEM((B, H), jnp.float32),
            pltpu.VMEM((ZR, H), jnp.float32),
            pltpu.VMEM_SHARED((NPAD, H), jnp.float32),
            pltpu.SemaphoreType.DMA,
            pltpu.SemaphoreType.DMA,
            pltpu.SemaphoreType.DMA,
            pltpu.SemaphoreType.DMA,
            pltpu.SemaphoreType.DMA,
            pltpu.SemaphoreType.DMA,
            pltpu.SemaphoreType.DMA,
            pltpu.SemaphoreType.DMA,
        ],
    )
    def sc_kern(xrel_hbm, cidx_hbm, dst_hbm, out_hbm,
                cb0, cb1, dstb0, dstb1, rows0, rows1, zb, acc,
                si0, si1, sd0, sd1, sg0, sg1, ss0, ss1):
        cid = lax.axis_index("c")
        sid = lax.axis_index("s")
        wid = cid * NS + sid

        cb = (cb0, cb1)
        dstb = (dstb0, dstb1)
        rows = (rows0, rows1)
        si = (si0, si1)
        sd = (sd0, sd1)
        sg = (sg0, sg1)
        ss = (ss0, ss1)

        def zrow(i, _):
            for j in range(H // 16):
                zb[i, pl.ds(j * 16, 16)] = jnp.zeros((16,), jnp.float32)
            return 0
        lax.fori_loop(0, ZR, zrow, 0)

        abase = sid * RPT
        def zacc(k, _):
            pltpu.sync_copy(zb, acc.at[pl.ds(abase + k * ZR, ZR)])
            return 0
        lax.fori_loop(0, RPT // ZR, zacc, 0)
        plsc.subcore_barrier()

        def fire_idx(t, p):
            pltpu.async_copy(cidx_hbm.at[wid, t], cb[p], si[p])

        def wait_idx(p):
            pltpu.make_async_copy(cidx_hbm.at[wid, 0], cb[p], si[p]).wait()

        def fire_dst(t, p):
            pltpu.async_copy(dst_hbm.at[wid, t], dstb[p], sd[p])

        def wait_dst(p):
            pltpu.make_async_copy(dst_hbm.at[wid, 0], dstb[p], sd[p]).wait()

        def fire_gather(p):
            pltpu.async_copy(xrel_hbm.at[cb[p]], rows[p], sg[p])

        def wait_gather(p):
            pltpu.make_async_copy(xrel_hbm.at[cb[p]], rows[p], sg[p]).wait()

        def fire_scatter(p):
            pltpu.async_copy(rows[p], acc.at[dstb[p].at[0]], ss[p], add=True)

        def wait_scatter(p):
            pltpu.make_async_copy(rows[p], acc.at[dstb[p].at[0]], ss[p]).wait()

        fire_idx(0, 0)
        fire_dst(0, 0)
        fire_idx(1, 1)
        fire_dst(1, 1)
        wait_idx(0)
        fire_gather(0)
        wait_idx(1)
        fire_gather(1)

        def body(g, _):
            t0 = 2 * g - 2
            wait_gather(0)
            fire_idx(t0 + 2, 0)
            wait_dst(0)
            fire_scatter(0)
            wait_gather(1)
            fire_idx(t0 + 3, 1)
            wait_dst(1)
            fire_scatter(1)
            wait_scatter(0)
            fire_dst(t0 + 2, 0)
            wait_idx(0)
            fire_gather(0)
            wait_scatter(1)
            fire_dst(t0 + 3, 1)
            wait_idx(1)
            fire_gather(1)
            return 0
        lax.fori_loop(1, NB // 2, body, 0)

        wait_gather(0)
        wait_dst(0)
        fire_scatter(0)
        wait_gather(1)
        wait_dst(1)
        fire_scatter(1)
        wait_scatter(0)
        wait_scatter(1)

        plsc.subcore_barrier()

        def ocp(k, _):
            r0 = pl.multiple_of(abase + k * ZR, 8)
            pltpu.sync_copy(acc.at[pl.ds(r0, ZR)], zb)
            pltpu.sync_copy(zb, out_hbm.at[cid, pl.ds(r0, ZR)])
            return 0
        lax.fori_loop(0, RPT // ZR, ocp, 0)

    return sc_kern(xrel, cidx3, dst4)


def _gru_body(x_ref, p_ref, wiT_ref, whT_ref, bi_ref, bh_ref, o_ref):
    red = p_ref[0] + p_ref[1]
    gi = jnp.dot(red, wiT_ref[:], preferred_element_type=jnp.float32) + bi_ref[:]
    gh = jnp.dot(x_ref[:], whT_ref[:], preferred_element_type=jnp.float32) + bh_ref[:]
    r = jax.nn.sigmoid(gi[:, :H] + gh[:, :H])
    z = jax.nn.sigmoid(gi[:, H:2 * H] + gh[:, H:2 * H])
    n = jnp.tanh(gi[:, 2 * H:] + r * gh[:, 2 * H:])
    o_ref[:] = (1.0 - z) * n + z * x_ref[:]


def _gru(x, parts, W_ih, W_hh, b_ih, b_hh):
    BN = 1000
    grid = (N // BN,)
    return pl.pallas_call(
        _gru_body,
        grid=grid,
        in_specs=[
            pl.BlockSpec((BN, H), lambda i: (i, 0)),
            pl.BlockSpec((NC, BN, H), lambda i: (0, i, 0)),
            pl.BlockSpec((H, 3 * H), lambda i: (0, 0)),
            pl.BlockSpec((H, 3 * H), lambda i: (0, 0)),
            pl.BlockSpec((1, 3 * H), lambda i: (0, 0)),
            pl.BlockSpec((1, 3 * H), lambda i: (0, 0)),
        ],
        out_specs=pl.BlockSpec((BN, H), lambda i: (i, 0)),
        out_shape=jax.ShapeDtypeStruct((N, H), jnp.float32),
    )(x, parts, W_ih.T, W_hh.T, b_ih.reshape(1, -1), b_hh.reshape(1, -1))


def kernel(x, edge_index, edge_type, relvectors, W_ih, W_hh, b_ih, b_hh):
    xrel = _expand(x, relvectors)
    cidx = _cidx(edge_index[0].reshape(E // 128, 128),
                 edge_type.reshape(E // 128, 128))
    cidx3 = cidx.reshape(NW, NB, B)
    dst4 = edge_index[1].reshape(NW, NB, 1, B)
    parts = _sc_segment_sum(xrel, cidx3, dst4)
    return _gru(x, parts, W_ih, W_hh, b_ih, b_hh)


# speedup vs baseline: 10.8845x; 1.1590x over previous
import functools

import jax
import jax.numpy as jnp
from jax import lax
from jax.experimental import pallas as pl
from jax.experimental.pallas import tpu as pltpu
from jax.experimental.pallas import tpu_sc as plsc

N = 10000
E = 320000
H = 128
R = 16

NC = 2
NS = 16
NW = NC * NS

EPW = E // NW
B = 80
NB = EPW // B
NPAD = 10240
RPT = NPAD // NS
ZR = 64


def _expand_body(x_ref, rel_ref, o_ref):
    bn = x_ref.shape[0]
    msg = x_ref[:][:, None, :] + rel_ref[:][None, :, :]
    o_ref[:] = msg.reshape(bn * R, H)


def _expand(x, relvec):
    BNE = 1000
    return pl.pallas_call(
        _expand_body,
        grid=(N // BNE,),
        in_specs=[
            pl.BlockSpec((BNE, H), lambda i: (i, 0)),
            pl.BlockSpec((R, H), lambda i: (0, 0)),
        ],
        out_specs=pl.BlockSpec((BNE * R, H), lambda i: (i, 0)),
        out_shape=jax.ShapeDtypeStruct((N * R, H), jnp.float32),
    )(x, relvec)


def _cidx_body(s_ref, t_ref, o_ref):
    o_ref[:] = s_ref[:] * R + t_ref[:]


def _cidx(src2, et2):
    ROWS = E // 128
    BR = ROWS
    return pl.pallas_call(
        _cidx_body,
        grid=(ROWS // BR,),
        in_specs=[
            pl.BlockSpec((BR, 128), lambda i: (i, 0)),
            pl.BlockSpec((BR, 128), lambda i: (i, 0)),
        ],
        out_specs=pl.BlockSpec((BR, 128), lambda i: (i, 0)),
        out_shape=jax.ShapeDtypeStruct((ROWS, 128), jnp.int32),
    )(src2, et2)


def _sc_segment_sum(xrel, cidx4, dst4):
    mesh = plsc.VectorSubcoreMesh(core_axis_name="c", subcore_axis_name="s")

    @functools.partial(
        pl.kernel,
        mesh=mesh,
        out_type=jax.ShapeDtypeStruct((NC, NPAD, H), jnp.float32),
        scratch_types=[
            pltpu.VMEM((1, B), jnp.int32),
            pltpu.VMEM((1, B), jnp.int32),
            pltpu.VMEM((1, B), jnp.int32),
            pltpu.VMEM((1, B), jnp.int32),
            pltpu.VMEM((B, H), jnp.float32),
            pltpu.VMEM((B, H), jnp.float32),
            pltpu.VMEM((ZR, H), jnp.float32),
            pltpu.VMEM_SHARED((NPAD, H), jnp.float32),
            pltpu.SemaphoreType.DMA,
            pltpu.SemaphoreType.DMA,
            pltpu.SemaphoreType.DMA,
            pltpu.SemaphoreType.DMA,
            pltpu.SemaphoreType.DMA,
            pltpu.SemaphoreType.DMA,
            pltpu.SemaphoreType.DMA,
            pltpu.SemaphoreType.DMA,
        ],
    )
    def sc_kern(xrel_hbm, cidx_hbm, dst_hbm, out_hbm,
                cb0, cb1, dstb0, dstb1, rows0, rows1, zb, acc,
                si0, si1, sd0, sd1, sg0, sg1, ss0, ss1):
        cid = lax.axis_index("c")
        sid = lax.axis_index("s")
        wid = cid * NS + sid

        cb = (cb0, cb1)
        dstb = (dstb0, dstb1)
        rows = (rows0, rows1)
        si = (si0, si1)
        sd = (sd0, sd1)
        sg = (sg0, sg1)
        ss = (ss0, ss1)

        def zrow(i, _):
            for j in range(H // 16):
                zb[i, pl.ds(j * 16, 16)] = jnp.zeros((16,), jnp.float32)
            return 0
        lax.fori_loop(0, ZR, zrow, 0)

        abase = sid * RPT
        def zacc(k, _):
            pltpu.sync_copy(zb, acc.at[pl.ds(abase + k * ZR, ZR)])
            return 0
        lax.fori_loop(0, RPT // ZR, zacc, 0)
        plsc.subcore_barrier()

        def fire_idx(t, p):
            pltpu.async_copy(cidx_hbm.at[wid, t], cb[p], si[p])

        def wait_idx(p):
            pltpu.make_async_copy(cidx_hbm.at[wid, 0], cb[p], si[p]).wait()

        def fire_dst(t, p):
            pltpu.async_copy(dst_hbm.at[wid, t], dstb[p], sd[p])

        def wait_dst(p):
            pltpu.make_async_copy(dst_hbm.at[wid, 0], dstb[p], sd[p]).wait()

        def fire_gather(p):
            pltpu.async_copy(xrel_hbm.at[cb[p].at[0]], rows[p], sg[p])

        def wait_gather(p):
            pltpu.make_async_copy(xrel_hbm.at[cb[p].at[0]], rows[p],
                                  sg[p]).wait()

        def fire_scatter(p):
            pltpu.async_copy(rows[p], acc.at[dstb[p].at[0]], ss[p], add=True)

        def wait_scatter(p):
            pltpu.make_async_copy(rows[p], acc.at[dstb[p].at[0]], ss[p]).wait()

        fire_idx(0, 0)
        fire_dst(0, 0)
        fire_idx(1, 1)
        fire_dst(1, 1)
        wait_idx(0)
        fire_gather(0)
        wait_idx(1)
        fire_gather(1)

        def body(g, _):
            t0 = 2 * g - 2
            wait_gather(0)
            fire_idx(t0 + 2, 0)
            wait_dst(0)
            fire_scatter(0)
            wait_gather(1)
            fire_idx(t0 + 3, 1)
            wait_dst(1)
            fire_scatter(1)
            wait_scatter(0)
            fire_dst(t0 + 2, 0)
            wait_idx(0)
            fire_gather(0)
            wait_scatter(1)
            fire_dst(t0 + 3, 1)
            wait_idx(1)
            fire_gather(1)
            return 0
        lax.fori_loop(1, (NB - 1) // 2, body, 0)

        wait_gather(0)
        wait_dst(0)
        fire_scatter(0)
        wait_gather(1)
        wait_dst(1)
        fire_scatter(1)
        wait_scatter(0)
        fire_idx(NB - 1, 0)
        fire_dst(NB - 1, 0)
        wait_idx(0)
        fire_gather(0)
        wait_gather(0)
        wait_dst(0)
        fire_scatter(0)
        wait_scatter(0)
        wait_scatter(1)

        plsc.subcore_barrier()

        def ocp(k, _):
            r0 = pl.multiple_of(abase + k * ZR, 8)
            pltpu.sync_copy(acc.at[pl.ds(r0, ZR)], zb)
            pltpu.sync_copy(zb, out_hbm.at[cid, pl.ds(r0, ZR)])
            return 0
        lax.fori_loop(0, RPT // ZR, ocp, 0)

    return sc_kern(xrel, cidx4, dst4)


def _gru_body(x_ref, p_ref, wiT_ref, whT_ref, bi_ref, bh_ref, o_ref):
    red = p_ref[0] + p_ref[1]
    gi = jnp.dot(red, wiT_ref[:], preferred_element_type=jnp.float32) + bi_ref[:]
    gh = jnp.dot(x_ref[:], whT_ref[:], preferred_element_type=jnp.float32) + bh_ref[:]
    r = jax.nn.sigmoid(gi[:, :H] + gh[:, :H])
    z = jax.nn.sigmoid(gi[:, H:2 * H] + gh[:, H:2 * H])
    n = jnp.tanh(gi[:, 2 * H:] + r * gh[:, 2 * H:])
    o_ref[:] = (1.0 - z) * n + z * x_ref[:]


def _gru(x, parts, W_ih, W_hh, b_ih, b_hh):
    BN = 1000
    grid = (N // BN,)
    return pl.pallas_call(
        _gru_body,
        grid=grid,
        in_specs=[
            pl.BlockSpec((BN, H), lambda i: (i, 0)),
            pl.BlockSpec((NC, BN, H), lambda i: (0, i, 0)),
            pl.BlockSpec((H, 3 * H), lambda i: (0, 0)),
            pl.BlockSpec((H, 3 * H), lambda i: (0, 0)),
            pl.BlockSpec((1, 3 * H), lambda i: (0, 0)),
            pl.BlockSpec((1, 3 * H), lambda i: (0, 0)),
        ],
        out_specs=pl.BlockSpec((BN, H), lambda i: (i, 0)),
        out_shape=jax.ShapeDtypeStruct((N, H), jnp.float32),
    )(x, parts, W_ih.T, W_hh.T, b_ih.reshape(1, -1), b_hh.reshape(1, -1))


def kernel(x, edge_index, edge_type, relvectors, W_ih, W_hh, b_ih, b_hh):
    xrel = _expand(x, relvectors)
    cidx = _cidx(edge_index[0].reshape(E // 128, 128),
                 edge_type.reshape(E // 128, 128))
    cidx4 = cidx.reshape(NW, NB, 1, B)
    dst4 = edge_index[1].reshape(NW, NB, 1, B)
    parts = _sc_segment_sum(xrel, cidx4, dst4)
    return _gru(x, parts, W_ih, W_hh, b_ih, b_hh)


# speedup vs baseline: 12.0053x; 1.1030x over previous
import functools

import jax
import jax.numpy as jnp
from jax import lax
from jax.experimental import pallas as pl
from jax.experimental.pallas import tpu as pltpu
from jax.experimental.pallas import tpu_sc as plsc

N = 10000
E = 320000
H = 128
R = 16

NC = 2
NS = 16
NW = NC * NS

B = 128
NB = 80
E2 = NW * NB * B
NPAD = 10240
RPT = NPAD // NS
ZR = 64


def _expand_body(x_ref, rel_ref, o_ref):
    bn = x_ref.shape[0]
    msg = x_ref[:][:, None, :] + rel_ref[:][None, :, :]
    o_ref[:] = msg.reshape(bn * R, H)


def _expand(x, relvec):
    BNE = 1000
    return pl.pallas_call(
        _expand_body,
        grid=(N // BNE,),
        in_specs=[
            pl.BlockSpec((BNE, H), lambda i: (i, 0)),
            pl.BlockSpec((R, H), lambda i: (0, 0)),
        ],
        out_specs=pl.BlockSpec((BNE * R, H), lambda i: (i, 0)),
        out_shape=jax.ShapeDtypeStruct((N * R, H), jnp.float32),
    )(x, relvec)


def _cidx_body(s_ref, t_ref, o_ref):
    o_ref[:] = s_ref[:] * R + t_ref[:]


def _cidx(src2, et2):
    ROWS = E2 // 128
    return pl.pallas_call(
        _cidx_body,
        grid=(1,),
        in_specs=[
            pl.BlockSpec((ROWS, 128), lambda i: (0, 0)),
            pl.BlockSpec((ROWS, 128), lambda i: (0, 0)),
        ],
        out_specs=pl.BlockSpec((ROWS, 128), lambda i: (0, 0)),
        out_shape=jax.ShapeDtypeStruct((ROWS, 128), jnp.int32),
    )(src2, et2)


def _sc_segment_sum(xrel, cidx4, dst4):
    mesh = plsc.VectorSubcoreMesh(core_axis_name="c", subcore_axis_name="s")

    @functools.partial(
        pl.kernel,
        mesh=mesh,
        out_type=jax.ShapeDtypeStruct((NC, NPAD, H), jnp.float32),
        scratch_types=[
            pltpu.VMEM((1, B), jnp.int32),
            pltpu.VMEM((1, B), jnp.int32),
            pltpu.VMEM((1, B), jnp.int32),
            pltpu.VMEM((1, B), jnp.int32),
            pltpu.VMEM((B, H), jnp.float32),
            pltpu.VMEM((B, H), jnp.float32),
            pltpu.VMEM((ZR, H), jnp.float32),
            pltpu.VMEM_SHARED((NPAD, H), jnp.float32),
            pltpu.SemaphoreType.DMA,
            pltpu.SemaphoreType.DMA,
            pltpu.SemaphoreType.DMA,
            pltpu.SemaphoreType.DMA,
            pltpu.SemaphoreType.DMA,
            pltpu.SemaphoreType.DMA,
            pltpu.SemaphoreType.DMA,
            pltpu.SemaphoreType.DMA,
        ],
    )
    def sc_kern(xrel_hbm, cidx_hbm, dst_hbm, out_hbm,
                cb0, cb1, dstb0, dstb1, rows0, rows1, zb, acc,
                si0, si1, sd0, sd1, sg0, sg1, ss0, ss1):
        cid = lax.axis_index("c")
        sid = lax.axis_index("s")
        wid = cid * NS + sid

        cb = (cb0, cb1)
        dstb = (dstb0, dstb1)
        rows = (rows0, rows1)
        si = (si0, si1)
        sd = (sd0, sd1)
        sg = (sg0, sg1)
        ss = (ss0, ss1)

        def zrow(i, _):
            for j in range(H // 16):
                zb[i, pl.ds(j * 16, 16)] = jnp.zeros((16,), jnp.float32)
            return 0
        lax.fori_loop(0, ZR, zrow, 0)

        abase = sid * RPT
        def zacc(k, _):
            pltpu.sync_copy(zb, acc.at[pl.ds(abase + k * ZR, ZR)])
            return 0
        lax.fori_loop(0, RPT // ZR, zacc, 0)
        plsc.subcore_barrier()

        def fire_idx(t, p):
            pltpu.async_copy(cidx_hbm.at[wid, t], cb[p], si[p])

        def wait_idx(p):
            pltpu.make_async_copy(cidx_hbm.at[wid, 0], cb[p], si[p]).wait()

        def fire_dst(t, p):
            pltpu.async_copy(dst_hbm.at[wid, t], dstb[p], sd[p])

        def wait_dst(p):
            pltpu.make_async_copy(dst_hbm.at[wid, 0], dstb[p], sd[p]).wait()

        def fire_gather(p):
            pltpu.async_copy(xrel_hbm.at[cb[p].at[0]], rows[p], sg[p])

        def wait_gather(p):
            pltpu.make_async_copy(xrel_hbm.at[cb[p].at[0]], rows[p],
                                  sg[p]).wait()

        def fire_scatter(p):
            pltpu.async_copy(rows[p], acc.at[dstb[p].at[0]], ss[p], add=True)

        def wait_scatter(p):
            pltpu.make_async_copy(rows[p], acc.at[dstb[p].at[0]], ss[p]).wait()

        fire_idx(0, 0)
        fire_dst(0, 0)
        fire_idx(1, 1)
        fire_dst(1, 1)
        wait_idx(0)
        fire_gather(0)
        wait_idx(1)
        fire_gather(1)

        def body(g, _):
            t0 = 2 * g - 2
            wait_gather(0)
            fire_idx(t0 + 2, 0)
            wait_dst(0)
            fire_scatter(0)
            wait_gather(1)
            fire_idx(t0 + 3, 1)
            wait_dst(1)
            fire_scatter(1)
            wait_scatter(0)
            fire_dst(t0 + 2, 0)
            wait_idx(0)
            fire_gather(0)
            wait_scatter(1)
            fire_dst(t0 + 3, 1)
            wait_idx(1)
            fire_gather(1)
            return 0
        lax.fori_loop(1, NB // 2, body, 0)

        wait_gather(0)
        wait_dst(0)
        fire_scatter(0)
        wait_gather(1)
        wait_dst(1)
        fire_scatter(1)
        wait_scatter(0)
        wait_scatter(1)

        plsc.subcore_barrier()

        def ocp(k, _):
            r0 = pl.multiple_of(abase + k * ZR, 8)
            pltpu.sync_copy(acc.at[pl.ds(r0, ZR)], zb)
            pltpu.sync_copy(zb, out_hbm.at[cid, pl.ds(r0, ZR)])
            return 0
        lax.fori_loop(0, RPT // ZR, ocp, 0)

    return sc_kern(xrel, cidx4, dst4)


def _gru_body(x_ref, p_ref, wiT_ref, whT_ref, bi_ref, bh_ref, o_ref):
    red = p_ref[0] + p_ref[1]
    gi = jnp.dot(red, wiT_ref[:], preferred_element_type=jnp.float32) + bi_ref[:]
    gh = jnp.dot(x_ref[:], whT_ref[:], preferred_element_type=jnp.float32) + bh_ref[:]
    r = jax.nn.sigmoid(gi[:, :H] + gh[:, :H])
    z = jax.nn.sigmoid(gi[:, H:2 * H] + gh[:, H:2 * H])
    n = jnp.tanh(gi[:, 2 * H:] + r * gh[:, 2 * H:])
    o_ref[:] = (1.0 - z) * n + z * x_ref[:]


def _gru(x, parts, W_ih, W_hh, b_ih, b_hh):
    BN = 1000
    grid = (N // BN,)
    return pl.pallas_call(
        _gru_body,
        grid=grid,
        in_specs=[
            pl.BlockSpec((BN, H), lambda i: (i, 0)),
            pl.BlockSpec((NC, BN, H), lambda i: (0, i, 0)),
            pl.BlockSpec((H, 3 * H), lambda i: (0, 0)),
            pl.BlockSpec((H, 3 * H), lambda i: (0, 0)),
            pl.BlockSpec((1, 3 * H), lambda i: (0, 0)),
            pl.BlockSpec((1, 3 * H), lambda i: (0, 0)),
        ],
        out_specs=pl.BlockSpec((BN, H), lambda i: (i, 0)),
        out_shape=jax.ShapeDtypeStruct((N, H), jnp.float32),
    )(x, parts, W_ih.T, W_hh.T, b_ih.reshape(1, -1), b_hh.reshape(1, -1))


def kernel(x, edge_index, edge_type, relvectors, W_ih, W_hh, b_ih, b_hh):
    xrel = _expand(x, relvectors)
    npad_e = E2 - E
    fake = jnp.arange(npad_e, dtype=jnp.int32)
    src_p = jnp.concatenate([edge_index[0], fake % N])
    et_p = jnp.concatenate([edge_type, fake % R])
    dst_p = jnp.concatenate([edge_index[1], N + fake % (NPAD - N)])
    cidx4 = _cidx(src_p.reshape(E2 // 128, 128),
                  et_p.reshape(E2 // 128, 128)).reshape(NW, NB, 1, B)
    dst4 = dst_p.reshape(NW, NB, 1, B)
    parts = _sc_segment_sum(xrel, cidx4, dst4)
    return _gru(x, parts, W_ih, W_hh, b_ih, b_hh)


# speedup vs baseline: 12.1930x; 1.0156x over previous
import functools

import jax
import jax.numpy as jnp
from jax import lax
from jax.experimental import pallas as pl
from jax.experimental.pallas import tpu as pltpu
from jax.experimental.pallas import tpu_sc as plsc

N = 10000
E = 320000
H = 128
R = 16

NC = 2
NS = 16
NW = NC * NS

B = 128
NB = 80
E2 = NW * NB * B
NPAD = 10240
RPT = NPAD // NS
ZR = 64


BNE = 1000
BRE = E2 // 128 // 10
RE = E // 128


def _prep_body(x_ref, rel_ref, s_ref, t_ref, d_ref, xr_ref, ci_ref, dp_ref):
    msg = x_ref[:][:, None, :] + rel_ref[:][None, :, :]
    xr_ref[:] = msg.reshape(BNE * R, H)
    i = pl.program_id(0)
    gid = ((i * BRE + lax.broadcasted_iota(jnp.int32, (BRE, 128), 0)) * 128
           + lax.broadcasted_iota(jnp.int32, (BRE, 128), 1))
    valid = gid < E
    ci_ref[:] = jnp.where(valid, s_ref[:] * R + t_ref[:], gid % (N * R))
    dp_ref[:] = jnp.where(valid, d_ref[:], N + gid % (NPAD - N))


def _prep(x, relvec, src2, et2, dst2):
    return pl.pallas_call(
        _prep_body,
        grid=(10,),
        in_specs=[
            pl.BlockSpec((BNE, H), lambda i: (i, 0)),
            pl.BlockSpec((R, H), lambda i: (0, 0)),
            pl.BlockSpec((BRE, 128), lambda i: (i, 0)),
            pl.BlockSpec((BRE, 128), lambda i: (i, 0)),
            pl.BlockSpec((BRE, 128), lambda i: (i, 0)),
        ],
        out_specs=[
            pl.BlockSpec((BNE * R, H), lambda i: (i, 0)),
            pl.BlockSpec((BRE, 128), lambda i: (i, 0)),
            pl.BlockSpec((BRE, 128), lambda i: (i, 0)),
        ],
        out_shape=[
            jax.ShapeDtypeStruct((N * R, H), jnp.float32),
            jax.ShapeDtypeStruct((E2 // 128, 128), jnp.int32),
            jax.ShapeDtypeStruct((E2 // 128, 128), jnp.int32),
        ],
    )(x, relvec, src2, et2, dst2)


def _sc_segment_sum(xrel, cidx4, dst4):
    mesh = plsc.VectorSubcoreMesh(core_axis_name="c", subcore_axis_name="s")

    @functools.partial(
        pl.kernel,
        mesh=mesh,
        out_type=jax.ShapeDtypeStruct((NC, NPAD, H), jnp.float32),
        scratch_types=[
            pltpu.VMEM((1, B), jnp.int32),
            pltpu.VMEM((1, B), jnp.int32),
            pltpu.VMEM((1, B), jnp.int32),
            pltpu.VMEM((1, B), jnp.int32),
            pltpu.VMEM((B, H), jnp.float32),
            pltpu.VMEM((B, H), jnp.float32),
            pltpu.VMEM((ZR, H), jnp.float32),
            pltpu.VMEM_SHARED((NPAD, H), jnp.float32),
            pltpu.SemaphoreType.DMA,
            pltpu.SemaphoreType.DMA,
            pltpu.SemaphoreType.DMA,
            pltpu.SemaphoreType.DMA,
            pltpu.SemaphoreType.DMA,
            pltpu.SemaphoreType.DMA,
            pltpu.SemaphoreType.DMA,
            pltpu.SemaphoreType.DMA,
        ],
    )
    def sc_kern(xrel_hbm, cidx_hbm, dst_hbm, out_hbm,
                cb0, cb1, dstb0, dstb1, rows0, rows1, zb, acc,
                si0, si1, sd0, sd1, sg0, sg1, ss0, ss1):
        cid = lax.axis_index("c")
        sid = lax.axis_index("s")
        wid = cid * NS + sid

        cb = (cb0, cb1)
        dstb = (dstb0, dstb1)
        rows = (rows0, rows1)
        si = (si0, si1)
        sd = (sd0, sd1)
        sg = (sg0, sg1)
        ss = (ss0, ss1)

        def zrow(i, _):
            for j in range(H // 16):
                zb[i, pl.ds(j * 16, 16)] = jnp.zeros((16,), jnp.float32)
            return 0
        lax.fori_loop(0, ZR, zrow, 0)

        abase = sid * RPT
        def zacc(k, _):
            pltpu.sync_copy(zb, acc.at[pl.ds(abase + k * ZR, ZR)])
            return 0
        lax.fori_loop(0, RPT // ZR, zacc, 0)
        plsc.subcore_barrier()

        def fire_idx(t, p):
            pltpu.async_copy(cidx_hbm.at[wid, t], cb[p], si[p])

        def wait_idx(p):
            pltpu.make_async_copy(cidx_hbm.at[wid, 0], cb[p], si[p]).wait()

        def fire_dst(t, p):
            pltpu.async_copy(dst_hbm.at[wid, t], dstb[p], sd[p])

        def wait_dst(p):
            pltpu.make_async_copy(dst_hbm.at[wid, 0], dstb[p], sd[p]).wait()

        def fire_gather(p):
            pltpu.async_copy(xrel_hbm.at[cb[p].at[0]], rows[p], sg[p])

        def wait_gather(p):
            pltpu.make_async_copy(xrel_hbm.at[cb[p].at[0]], rows[p],
                                  sg[p]).wait()

        def fire_scatter(p):
            pltpu.async_copy(rows[p], acc.at[dstb[p].at[0]], ss[p], add=True)

        def wait_scatter(p):
            pltpu.make_async_copy(rows[p], acc.at[dstb[p].at[0]], ss[p]).wait()

        fire_idx(0, 0)
        fire_dst(0, 0)
        fire_idx(1, 1)
        fire_dst(1, 1)
        wait_idx(0)
        fire_gather(0)
        wait_idx(1)
        fire_gather(1)

        def body(g, _):
            t0 = 2 * g - 2
            wait_gather(0)
            fire_idx(t0 + 2, 0)
            wait_dst(0)
            fire_scatter(0)
            wait_gather(1)
            fire_idx(t0 + 3, 1)
            wait_dst(1)
            fire_scatter(1)
            wait_scatter(0)
            fire_dst(t0 + 2, 0)
            wait_idx(0)
            fire_gather(0)
            wait_scatter(1)
            fire_dst(t0 + 3, 1)
            wait_idx(1)
            fire_gather(1)
            return 0
        lax.fori_loop(1, NB // 2, body, 0)

        wait_gather(0)
        wait_dst(0)
        fire_scatter(0)
        wait_gather(1)
        wait_dst(1)
        fire_scatter(1)
        wait_scatter(0)
        wait_scatter(1)

        plsc.subcore_barrier()

        def ocp(k, _):
            r0 = pl.multiple_of(abase + k * ZR, 8)
            pltpu.sync_copy(acc.at[pl.ds(r0, ZR)], zb)
            pltpu.sync_copy(zb, out_hbm.at[cid, pl.ds(r0, ZR)])
            return 0
        lax.fori_loop(0, RPT // ZR, ocp, 0)

    return sc_kern(xrel, cidx4, dst4)


def _gru_body(x_ref, p_ref, wiT_ref, whT_ref, bi_ref, bh_ref, o_ref):
    red = p_ref[0] + p_ref[1]
    gi = jnp.dot(red, wiT_ref[:], preferred_element_type=jnp.float32) + bi_ref[:]
    gh = jnp.dot(x_ref[:], whT_ref[:], preferred_element_type=jnp.float32) + bh_ref[:]
    r = jax.nn.sigmoid(gi[:, :H] + gh[:, :H])
    z = jax.nn.sigmoid(gi[:, H:2 * H] + gh[:, H:2 * H])
    n = jnp.tanh(gi[:, 2 * H:] + r * gh[:, 2 * H:])
    o_ref[:] = (1.0 - z) * n + z * x_ref[:]


def _gru(x, parts, W_ih, W_hh, b_ih, b_hh):
    BN = 1000
    grid = (N // BN,)
    return pl.pallas_call(
        _gru_body,
        grid=grid,
        in_specs=[
            pl.BlockSpec((BN, H), lambda i: (i, 0)),
            pl.BlockSpec((NC, BN, H), lambda i: (0, i, 0)),
            pl.BlockSpec((H, 3 * H), lambda i: (0, 0)),
            pl.BlockSpec((H, 3 * H), lambda i: (0, 0)),
            pl.BlockSpec((1, 3 * H), lambda i: (0, 0)),
            pl.BlockSpec((1, 3 * H), lambda i: (0, 0)),
        ],
        out_specs=pl.BlockSpec((BN, H), lambda i: (i, 0)),
        out_shape=jax.ShapeDtypeStruct((N, H), jnp.float32),
    )(x, parts, W_ih.T, W_hh.T, b_ih.reshape(1, -1), b_hh.reshape(1, -1))


def kernel(x, edge_index, edge_type, relvectors, W_ih, W_hh, b_ih, b_hh):
    src2 = edge_index[0].reshape(RE, 128)
    et2 = edge_type.reshape(RE, 128)
    dst2 = edge_index[1].reshape(RE, 128)
    xrel, cidx, dstp = _prep(x, relvectors, src2, et2, dst2)
    cidx4 = cidx.reshape(NW, NB, 1, B)
    dst4 = dstp.reshape(NW, NB, 1, B)
    parts = _sc_segment_sum(xrel, cidx4, dst4)
    return _gru(x, parts, W_ih, W_hh, b_ih, b_hh)


# speedup vs baseline: 12.9472x; 1.0619x over previous
import functools

import jax
import jax.numpy as jnp
from jax import lax
from jax.experimental import pallas as pl
from jax.experimental.pallas import tpu as pltpu
from jax.experimental.pallas import tpu_sc as plsc

N = 10000
E = 320000
H = 128
R = 16

NC = 2
NS = 16
NW = NC * NS

B = 128
NB = 80
E2 = NW * NB * B
NPAD = 10240
RPT = NPAD // NS
ZR = 64


BNE = 1000
BRE = E2 // 128 // 10
RE = E // 128


def _prep_body(x_ref, rel_ref, s_ref, t_ref, d_ref, xr_ref, ci_ref, dp_ref):
    msg = x_ref[:][:, None, :] + rel_ref[:][None, :, :]
    xr_ref[:] = msg.reshape(BNE * R, H)
    i = pl.program_id(0)
    gid = ((i * BRE + lax.broadcasted_iota(jnp.int32, (BRE, 128), 0)) * 128
           + lax.broadcasted_iota(jnp.int32, (BRE, 128), 1))
    valid = gid < E
    ci_ref[:] = jnp.where(valid, s_ref[:] * R + t_ref[:], gid % (N * R))
    dp_ref[:] = jnp.where(valid, d_ref[:], N + gid % (NPAD - N))


def _prep(x, relvec, src2, et2, dst2):
    return pl.pallas_call(
        _prep_body,
        grid=(10,),
        in_specs=[
            pl.BlockSpec((BNE, H), lambda i: (i, 0)),
            pl.BlockSpec((R, H), lambda i: (0, 0)),
            pl.BlockSpec((BRE, 128), lambda i: (i, 0)),
            pl.BlockSpec((BRE, 128), lambda i: (i, 0)),
            pl.BlockSpec((BRE, 128), lambda i: (i, 0)),
        ],
        out_specs=[
            pl.BlockSpec((BNE * R, H), lambda i: (i, 0)),
            pl.BlockSpec((BRE, 128), lambda i: (i, 0)),
            pl.BlockSpec((BRE, 128), lambda i: (i, 0)),
        ],
        out_shape=[
            jax.ShapeDtypeStruct((N * R, H), jnp.float32),
            jax.ShapeDtypeStruct((E2 // 128, 128), jnp.int32),
            jax.ShapeDtypeStruct((E2 // 128, 128), jnp.int32),
        ],
    )(x, relvec, src2, et2, dst2)


def _sc_segment_sum(xrel, cidx5, dst5):
    mesh = plsc.VectorSubcoreMesh(core_axis_name="c", subcore_axis_name="s")

    @functools.partial(
        pl.kernel,
        mesh=mesh,
        out_type=jax.ShapeDtypeStruct((NC, NPAD, H), jnp.float32),
        scratch_types=[
            pltpu.VMEM((8, B), jnp.int32),
            pltpu.VMEM((8, B), jnp.int32),
            pltpu.VMEM((8, B), jnp.int32),
            pltpu.VMEM((8, B), jnp.int32),
            pltpu.VMEM((B, H), jnp.float32),
            pltpu.VMEM((B, H), jnp.float32),
            pltpu.VMEM((ZR, H), jnp.float32),
            pltpu.VMEM_SHARED((NPAD, H), jnp.float32),
            pltpu.SemaphoreType.DMA,
            pltpu.SemaphoreType.DMA,
            pltpu.SemaphoreType.DMA,
            pltpu.SemaphoreType.DMA,
            pltpu.SemaphoreType.DMA,
            pltpu.SemaphoreType.DMA,
            pltpu.SemaphoreType.DMA,
            pltpu.SemaphoreType.DMA,
        ],
    )
    def sc_kern(xrel_hbm, cidx_hbm, dst_hbm, out_hbm,
                cb0, cb1, dstb0, dstb1, rows0, rows1, zb, acc,
                si0, si1, sd0, sd1, sg0, sg1, ss0, ss1):
        cid = lax.axis_index("c")
        sid = lax.axis_index("s")
        wid = cid * NS + sid

        cb = (cb0, cb1)
        dstb = (dstb0, dstb1)
        rows = (rows0, rows1)
        si = (si0, si1)
        sd = (sd0, sd1)
        sg = (sg0, sg1)
        ss = (ss0, ss1)

        def zrow(i, _):
            for j in range(H // 16):
                zb[i, pl.ds(j * 16, 16)] = jnp.zeros((16,), jnp.float32)
            return 0
        lax.fori_loop(0, ZR, zrow, 0)

        abase = sid * RPT
        def zacc(k, _):
            pltpu.sync_copy(zb, acc.at[pl.ds(abase + k * ZR, ZR)])
            return 0
        lax.fori_loop(0, RPT // ZR, zacc, 0)
        plsc.subcore_barrier()

        def fire_chunk(c, q):
            pltpu.async_copy(cidx_hbm.at[wid, c], cb[q], si[q])
            pltpu.async_copy(dst_hbm.at[wid, c], dstb[q], sd[q])

        def wait_chunk(q):
            pltpu.make_async_copy(cidx_hbm.at[wid, 0], cb[q], si[q]).wait()
            pltpu.make_async_copy(dst_hbm.at[wid, 0], dstb[q], sd[q]).wait()

        def fire_gather(t, p):
            q = (t >> 3) & 1
            pltpu.async_copy(xrel_hbm.at[cb[q].at[t & 7]], rows[p], sg[p])

        def wait_gather(t, p):
            q = (t >> 3) & 1
            pltpu.make_async_copy(xrel_hbm.at[cb[q].at[t & 7]], rows[p],
                                  sg[p]).wait()

        def fire_scatter(t, p):
            q = (t >> 3) & 1
            pltpu.async_copy(rows[p], acc.at[dstb[q].at[t & 7]], ss[p],
                             add=True)

        def wait_scatter(t, p):
            q = (t >> 3) & 1
            pltpu.make_async_copy(rows[p], acc.at[dstb[q].at[t & 7]],
                                  ss[p]).wait()

        NCH = NB // 8
        fire_chunk(0, 0)
        fire_chunk(1, 1)
        wait_chunk(0)
        fire_gather(0, 0)
        fire_gather(1, 1)

        for t in range(NB):
            p = t & 1
            wait_gather(t, p)
            fire_scatter(t, p)
            if t >= 1:
                wait_scatter(t - 1, p ^ 1)
                if t % 8 == 0 and t >= 8 and (t // 8 + 1) < NCH:
                    fire_chunk(t // 8 + 1, (t // 8 + 1) & 1)
            tn = t + 1
            if tn < NB and tn >= 2:
                if tn % 8 == 0:
                    wait_chunk((tn // 8) & 1)
                fire_gather(tn, tn & 1)
        wait_scatter(NB - 1, (NB - 1) & 1)

        plsc.subcore_barrier()

        def ocp(k, _):
            r0 = pl.multiple_of(abase + k * ZR, 8)
            pltpu.sync_copy(acc.at[pl.ds(r0, ZR)], zb)
            pltpu.sync_copy(zb, out_hbm.at[cid, pl.ds(r0, ZR)])
            return 0
        lax.fori_loop(0, RPT // ZR, ocp, 0)

    return sc_kern(xrel, cidx5, dst5)


def _gru_body(x_ref, p_ref, wiT_ref, whT_ref, bi_ref, bh_ref, o_ref):
    red = p_ref[0] + p_ref[1]
    gi = jnp.dot(red, wiT_ref[:], preferred_element_type=jnp.float32) + bi_ref[:]
    gh = jnp.dot(x_ref[:], whT_ref[:], preferred_element_type=jnp.float32) + bh_ref[:]
    r = jax.nn.sigmoid(gi[:, :H] + gh[:, :H])
    z = jax.nn.sigmoid(gi[:, H:2 * H] + gh[:, H:2 * H])
    n = jnp.tanh(gi[:, 2 * H:] + r * gh[:, 2 * H:])
    o_ref[:] = (1.0 - z) * n + z * x_ref[:]


def _gru(x, parts, W_ih, W_hh, b_ih, b_hh):
    BN = 1000
    grid = (N // BN,)
    return pl.pallas_call(
        _gru_body,
        grid=grid,
        in_specs=[
            pl.BlockSpec((BN, H), lambda i: (i, 0)),
            pl.BlockSpec((NC, BN, H), lambda i: (0, i, 0)),
            pl.BlockSpec((H, 3 * H), lambda i: (0, 0)),
            pl.BlockSpec((H, 3 * H), lambda i: (0, 0)),
            pl.BlockSpec((1, 3 * H), lambda i: (0, 0)),
            pl.BlockSpec((1, 3 * H), lambda i: (0, 0)),
        ],
        out_specs=pl.BlockSpec((BN, H), lambda i: (i, 0)),
        out_shape=jax.ShapeDtypeStruct((N, H), jnp.float32),
    )(x, parts, W_ih.T, W_hh.T, b_ih.reshape(1, -1), b_hh.reshape(1, -1))


def kernel(x, edge_index, edge_type, relvectors, W_ih, W_hh, b_ih, b_hh):
    src2 = edge_index[0].reshape(RE, 128)
    et2 = edge_type.reshape(RE, 128)
    dst2 = edge_index[1].reshape(RE, 128)
    xrel, cidx, dstp = _prep(x, relvectors, src2, et2, dst2)
    cidx5 = cidx.reshape(NW, NB // 8, 8, B)
    dst5 = dstp.reshape(NW, NB // 8, 8, B)
    parts = _sc_segment_sum(xrel, cidx5, dst5)
    return _gru(x, parts, W_ih, W_hh, b_ih, b_hh)


# speedup vs baseline: 12.9914x; 1.0034x over previous
import functools

import jax
import jax.numpy as jnp
from jax import lax
from jax.experimental import pallas as pl
from jax.experimental.pallas import tpu as pltpu
from jax.experimental.pallas import tpu_sc as plsc

N = 10000
E = 320000
H = 128
R = 16

NC = 2
NS = 16
NW = NC * NS

B = 128
NB = 80
E2 = NW * NB * B
NPAD = 10240
RPT = NPAD // NS
ZR = 64


BNE = 1000
BRE = E2 // 128 // 10
RE = E // 128


def _prep_body(x_ref, rel_ref, s_ref, t_ref, d_ref, xr_ref, ci_ref, dp_ref):
    msg = x_ref[:][:, None, :] + rel_ref[:][None, :, :]
    xr_ref[:] = msg.reshape(BNE * R, H)
    i = pl.program_id(0)
    gid = ((i * BRE + lax.broadcasted_iota(jnp.int32, (BRE, 128), 0)) * 128
           + lax.broadcasted_iota(jnp.int32, (BRE, 128), 1))
    valid = gid < E
    ci_ref[:] = jnp.where(valid, s_ref[:] * R + t_ref[:], gid % (N * R))
    dp_ref[:] = jnp.where(valid, d_ref[:], N + gid % (NPAD - N))


def _prep(x, relvec, src2, et2, dst2):
    return pl.pallas_call(
        _prep_body,
        grid=(10,),
        in_specs=[
            pl.BlockSpec((BNE, H), lambda i: (i, 0)),
            pl.BlockSpec((R, H), lambda i: (0, 0)),
            pl.BlockSpec((BRE, 128), lambda i: (i, 0)),
            pl.BlockSpec((BRE, 128), lambda i: (i, 0)),
            pl.BlockSpec((BRE, 128), lambda i: (i, 0)),
        ],
        out_specs=[
            pl.BlockSpec((BNE * R, H), lambda i: (i, 0)),
            pl.BlockSpec((BRE, 128), lambda i: (i, 0)),
            pl.BlockSpec((BRE, 128), lambda i: (i, 0)),
        ],
        out_shape=[
            jax.ShapeDtypeStruct((N * R, H), jnp.float32),
            jax.ShapeDtypeStruct((E2 // 128, 128), jnp.int32),
            jax.ShapeDtypeStruct((E2 // 128, 128), jnp.int32),
        ],
    )(x, relvec, src2, et2, dst2)


def _sc_segment_sum(xrel, cidx5, dst5):
    mesh = plsc.VectorSubcoreMesh(core_axis_name="c", subcore_axis_name="s")

    @functools.partial(
        pl.kernel,
        mesh=mesh,
        out_type=jax.ShapeDtypeStruct((NC, NPAD, H), jnp.float32),
        scratch_types=[
            pltpu.VMEM((8, B), jnp.int32),
            pltpu.VMEM((8, B), jnp.int32),
            pltpu.VMEM((8, B), jnp.int32),
            pltpu.VMEM((8, B), jnp.int32),
            pltpu.VMEM((B, H), jnp.float32),
            pltpu.VMEM((B, H), jnp.float32),
            pltpu.VMEM((ZR, H), jnp.float32),
            pltpu.VMEM_SHARED((NPAD, H), jnp.float32),
            pltpu.SemaphoreType.DMA,
            pltpu.SemaphoreType.DMA,
            pltpu.SemaphoreType.DMA,
            pltpu.SemaphoreType.DMA,
            pltpu.SemaphoreType.DMA,
            pltpu.SemaphoreType.DMA,
            pltpu.SemaphoreType.DMA,
            pltpu.SemaphoreType.DMA,
        ],
    )
    def sc_kern(xrel_hbm, cidx_hbm, dst_hbm, out_hbm,
                cb0, cb1, dstb0, dstb1, rows0, rows1, zb, acc,
                si0, si1, sd0, sd1, sg0, sg1, ss0, ss1):
        cid = lax.axis_index("c")
        sid = lax.axis_index("s")
        wid = cid * NS + sid

        cb = (cb0, cb1)
        dstb = (dstb0, dstb1)
        rows = (rows0, rows1)
        si = (si0, si1)
        sd = (sd0, sd1)
        sg = (sg0, sg1)
        ss = (ss0, ss1)

        def zrow(i, _):
            for j in range(H // 16):
                zb[i, pl.ds(j * 16, 16)] = jnp.zeros((16,), jnp.float32)
            return 0
        lax.fori_loop(0, ZR, zrow, 0)

        abase = sid * RPT
        def zacc(k, _):
            pltpu.sync_copy(zb, acc.at[pl.ds(abase + k * ZR, ZR)])
            return 0
        lax.fori_loop(0, RPT // ZR, zacc, 0)
        plsc.subcore_barrier()

        def fire_chunk(c, q):
            pltpu.async_copy(cidx_hbm.at[wid, c], cb[q], si[q])
            pltpu.async_copy(dst_hbm.at[wid, c], dstb[q], sd[q])

        def wait_chunk(q):
            pltpu.make_async_copy(cidx_hbm.at[wid, 0], cb[q], si[q]).wait()
            pltpu.make_async_copy(dst_hbm.at[wid, 0], dstb[q], sd[q]).wait()

        def fire_gather(t, p):
            q = (t >> 3) & 1
            pltpu.async_copy(xrel_hbm.at[cb[q].at[t & 7]], rows[p], sg[p])

        def wait_gather(t, p):
            q = (t >> 3) & 1
            pltpu.make_async_copy(xrel_hbm.at[cb[q].at[t & 7]], rows[p],
                                  sg[p]).wait()

        def fire_scatter(t, p):
            q = (t >> 3) & 1
            pltpu.async_copy(rows[p], acc.at[dstb[q].at[t & 7]], ss[p],
                             add=True)

        def wait_scatter(t, p):
            q = (t >> 3) & 1
            pltpu.make_async_copy(rows[p], acc.at[dstb[q].at[t & 7]],
                                  ss[p]).wait()

        NCH = NB // 8
        fire_chunk(0, 0)
        fire_chunk(1, 1)
        wait_chunk(0)
        fire_gather(0, 0)
        fire_gather(1, 1)

        for t in range(NB):
            p = t & 1
            wait_gather(t, p)
            fire_scatter(t, p)
            if t >= 1:
                wait_scatter(t - 1, p ^ 1)
                if t % 8 == 0 and t >= 8 and (t // 8 + 1) < NCH:
                    fire_chunk(t // 8 + 1, (t // 8 + 1) & 1)
            tn = t + 1
            if tn < NB and tn >= 2:
                if tn % 8 == 0:
                    wait_chunk((tn // 8) & 1)
                fire_gather(tn, tn & 1)
        wait_scatter(NB - 1, (NB - 1) & 1)

        plsc.subcore_barrier()

        r0 = pl.multiple_of(abase, 8)
        pltpu.sync_copy(acc.at[pl.ds(r0, RPT)],
                        out_hbm.at[cid, pl.ds(r0, RPT)])

    return sc_kern(xrel, cidx5, dst5)


def _gru_body(x_ref, p_ref, wiT_ref, whT_ref, bi_ref, bh_ref, o_ref):
    red = p_ref[0] + p_ref[1]
    gi = jnp.dot(red, wiT_ref[:], preferred_element_type=jnp.float32) + bi_ref[:]
    gh = jnp.dot(x_ref[:], whT_ref[:], preferred_element_type=jnp.float32) + bh_ref[:]
    r = jax.nn.sigmoid(gi[:, :H] + gh[:, :H])
    z = jax.nn.sigmoid(gi[:, H:2 * H] + gh[:, H:2 * H])
    n = jnp.tanh(gi[:, 2 * H:] + r * gh[:, 2 * H:])
    o_ref[:] = (1.0 - z) * n + z * x_ref[:]


def _gru(x, parts, W_ih, W_hh, b_ih, b_hh):
    BN = 1000
    grid = (N // BN,)
    return pl.pallas_call(
        _gru_body,
        grid=grid,
        in_specs=[
            pl.BlockSpec((BN, H), lambda i: (i, 0)),
            pl.BlockSpec((NC, BN, H), lambda i: (0, i, 0)),
            pl.BlockSpec((H, 3 * H), lambda i: (0, 0)),
            pl.BlockSpec((H, 3 * H), lambda i: (0, 0)),
            pl.BlockSpec((1, 3 * H), lambda i: (0, 0)),
            pl.BlockSpec((1, 3 * H), lambda i: (0, 0)),
        ],
        out_specs=pl.BlockSpec((BN, H), lambda i: (i, 0)),
        out_shape=jax.ShapeDtypeStruct((N, H), jnp.float32),
    )(x, parts, W_ih.T, W_hh.T, b_ih.reshape(1, -1), b_hh.reshape(1, -1))


def kernel(x, edge_index, edge_type, relvectors, W_ih, W_hh, b_ih, b_hh):
    src2 = edge_index[0].reshape(RE, 128)
    et2 = edge_type.reshape(RE, 128)
    dst2 = edge_index[1].reshape(RE, 128)
    xrel, cidx, dstp = _prep(x, relvectors, src2, et2, dst2)
    cidx5 = cidx.reshape(NW, NB // 8, 8, B)
    dst5 = dstp.reshape(NW, NB // 8, 8, B)
    parts = _sc_segment_sum(xrel, cidx5, dst5)
    return _gru(x, parts, W_ih, W_hh, b_ih, b_hh)


# speedup vs baseline: 13.0419x; 1.0039x over previous
import functools

import jax
import jax.numpy as jnp
from jax import lax
from jax.experimental import pallas as pl
from jax.experimental.pallas import tpu as pltpu
from jax.experimental.pallas import tpu_sc as plsc

N = 10000
E = 320000
H = 128
R = 16

NC = 2
NS = 16
NW = NC * NS

B = 128
NB = 80
E2 = NW * NB * B
NPAD = 10240
RPT = NPAD // NS
ZR = 64


BNE = 1000
BRE = E2 // 128 // 10
RE = E // 128


def _prep_body(x_ref, rel_ref, s_ref, t_ref, d_ref, xr_ref, ci_ref, dp_ref):
    msg = x_ref[:][:, None, :] + rel_ref[:][None, :, :]
    xr_ref[:] = msg.reshape(BNE * R, H)
    i = pl.program_id(0)
    gid = ((i * BRE + lax.broadcasted_iota(jnp.int32, (BRE, 128), 0)) * 128
           + lax.broadcasted_iota(jnp.int32, (BRE, 128), 1))
    valid = gid < E
    ci_ref[:] = jnp.where(valid, s_ref[:] * R + t_ref[:], gid % (N * R))
    dp_ref[:] = jnp.where(valid, d_ref[:], N + gid % (NPAD - N))


def _prep(x, relvec, src2, et2, dst2):
    return pl.pallas_call(
        _prep_body,
        grid=(10,),
        in_specs=[
            pl.BlockSpec((BNE, H), lambda i: (i, 0)),
            pl.BlockSpec((R, H), lambda i: (0, 0)),
            pl.BlockSpec((BRE, 128), lambda i: (i, 0)),
            pl.BlockSpec((BRE, 128), lambda i: (i, 0)),
            pl.BlockSpec((BRE, 128), lambda i: (i, 0)),
        ],
        out_specs=[
            pl.BlockSpec((BNE * R, H), lambda i: (i, 0)),
            pl.BlockSpec((BRE, 128), lambda i: (i, 0)),
            pl.BlockSpec((BRE, 128), lambda i: (i, 0)),
        ],
        out_shape=[
            jax.ShapeDtypeStruct((N * R, H), jnp.float32),
            jax.ShapeDtypeStruct((E2 // 128, 128), jnp.int32),
            jax.ShapeDtypeStruct((E2 // 128, 128), jnp.int32),
        ],
    )(x, relvec, src2, et2, dst2)


def _sc_segment_sum(xrel, cidx5, dst5):
    mesh = plsc.VectorSubcoreMesh(core_axis_name="c", subcore_axis_name="s")

    @functools.partial(
        pl.kernel,
        mesh=mesh,
        out_type=jax.ShapeDtypeStruct((NC, NPAD, H), jnp.float32),
        scratch_types=[
            pltpu.VMEM((8, B), jnp.int32),
            pltpu.VMEM((8, B), jnp.int32),
            pltpu.VMEM((8, B), jnp.int32),
            pltpu.VMEM((8, B), jnp.int32),
            pltpu.VMEM((B, H), jnp.float32),
            pltpu.VMEM((B, H), jnp.float32),
            pltpu.VMEM((ZR, H), jnp.float32),
            pltpu.VMEM_SHARED((NPAD, H), jnp.float32),
            pltpu.SemaphoreType.DMA,
            pltpu.SemaphoreType.DMA,
            pltpu.SemaphoreType.DMA,
            pltpu.SemaphoreType.DMA,
            pltpu.SemaphoreType.DMA,
            pltpu.SemaphoreType.DMA,
            pltpu.SemaphoreType.DMA,
            pltpu.SemaphoreType.DMA,
        ],
    )
    def sc_kern(xrel_hbm, cidx_hbm, dst_hbm, out_hbm,
                cb0, cb1, dstb0, dstb1, rows0, rows1, zb, acc,
                si0, si1, sd0, sd1, sg0, sg1, ss0, ss1):
        cid = lax.axis_index("c")
        sid = lax.axis_index("s")
        wid = cid * NS + sid

        cb = (cb0, cb1)
        dstb = (dstb0, dstb1)
        rows = (rows0, rows1)
        si = (si0, si1)
        sd = (sd0, sd1)
        sg = (sg0, sg1)
        ss = (ss0, ss1)

        def zrow(i, _):
            for j in range(H // 16):
                zb[i, pl.ds(j * 16, 16)] = jnp.zeros((16,), jnp.float32)
            return 0
        lax.fori_loop(0, ZR, zrow, 0)

        abase = sid * RPT
        for k in range(RPT // ZR):
            pltpu.async_copy(zb, acc.at[pl.ds(abase + k * ZR, ZR)], si0)
        for k in range(RPT // ZR):
            pltpu.make_async_copy(zb, acc.at[pl.ds(abase, ZR)], si0).wait()
        plsc.subcore_barrier()

        def fire_chunk(c, q):
            pltpu.async_copy(cidx_hbm.at[wid, c], cb[q], si[q])
            pltpu.async_copy(dst_hbm.at[wid, c], dstb[q], sd[q])

        def wait_chunk(q):
            pltpu.make_async_copy(cidx_hbm.at[wid, 0], cb[q], si[q]).wait()
            pltpu.make_async_copy(dst_hbm.at[wid, 0], dstb[q], sd[q]).wait()

        def fire_gather(t, p):
            q = (t >> 3) & 1
            pltpu.async_copy(xrel_hbm.at[cb[q].at[t & 7]], rows[p], sg[p])

        def wait_gather(t, p):
            q = (t >> 3) & 1
            pltpu.make_async_copy(xrel_hbm.at[cb[q].at[t & 7]], rows[p],
                                  sg[p]).wait()

        def fire_scatter(t, p):
            q = (t >> 3) & 1
            pltpu.async_copy(rows[p], acc.at[dstb[q].at[t & 7]], ss[p],
                             add=True)

        def wait_scatter(t, p):
            q = (t >> 3) & 1
            pltpu.make_async_copy(rows[p], acc.at[dstb[q].at[t & 7]],
                                  ss[p]).wait()

        NCH = NB // 8
        fire_chunk(0, 0)
        fire_chunk(1, 1)
        wait_chunk(0)
        fire_gather(0, 0)
        fire_gather(1, 1)

        for t in range(NB):
            p = t & 1
            wait_gather(t, p)
            fire_scatter(t, p)
            if t >= 1:
                wait_scatter(t - 1, p ^ 1)
                if t % 8 == 0 and t >= 8 and (t // 8 + 1) < NCH:
                    fire_chunk(t // 8 + 1, (t // 8 + 1) & 1)
            tn = t + 1
            if tn < NB and tn >= 2:
                if tn % 8 == 0:
                    wait_chunk((tn // 8) & 1)
                fire_gather(tn, tn & 1)
        wait_scatter(NB - 1, (NB - 1) & 1)

        plsc.subcore_barrier()

        r0 = pl.multiple_of(abase, 8)
        pltpu.sync_copy(acc.at[pl.ds(r0, RPT)],
                        out_hbm.at[cid, pl.ds(r0, RPT)])

    return sc_kern(xrel, cidx5, dst5)


def _gru_body(x_ref, p_ref, wiT_ref, whT_ref, bi_ref, bh_ref, o_ref):
    red = p_ref[0] + p_ref[1]
    gi = jnp.dot(red, wiT_ref[:], preferred_element_type=jnp.float32) + bi_ref[:]
    gh = jnp.dot(x_ref[:], whT_ref[:], preferred_element_type=jnp.float32) + bh_ref[:]
    r = jax.nn.sigmoid(gi[:, :H] + gh[:, :H])
    z = jax.nn.sigmoid(gi[:, H:2 * H] + gh[:, H:2 * H])
    n = jnp.tanh(gi[:, 2 * H:] + r * gh[:, 2 * H:])
    o_ref[:] = (1.0 - z) * n + z * x_ref[:]


def _gru(x, parts, W_ih, W_hh, b_ih, b_hh):
    BN = 1000
    grid = (N // BN,)
    return pl.pallas_call(
        _gru_body,
        grid=grid,
        in_specs=[
            pl.BlockSpec((BN, H), lambda i: (i, 0)),
            pl.BlockSpec((NC, BN, H), lambda i: (0, i, 0)),
            pl.BlockSpec((H, 3 * H), lambda i: (0, 0)),
            pl.BlockSpec((H, 3 * H), lambda i: (0, 0)),
            pl.BlockSpec((1, 3 * H), lambda i: (0, 0)),
            pl.BlockSpec((1, 3 * H), lambda i: (0, 0)),
        ],
        out_specs=pl.BlockSpec((BN, H), lambda i: (i, 0)),
        out_shape=jax.ShapeDtypeStruct((N, H), jnp.float32),
    )(x, parts, W_ih.T, W_hh.T, b_ih.reshape(1, -1), b_hh.reshape(1, -1))


def kernel(x, edge_index, edge_type, relvectors, W_ih, W_hh, b_ih, b_hh):
    src2 = edge_index[0].reshape(RE, 128)
    et2 = edge_type.reshape(RE, 128)
    dst2 = edge_index[1].reshape(RE, 128)
    xrel, cidx, dstp = _prep(x, relvectors, src2, et2, dst2)
    cidx5 = cidx.reshape(NW, NB // 8, 8, B)
    dst5 = dstp.reshape(NW, NB // 8, 8, B)
    parts = _sc_segment_sum(xrel, cidx5, dst5)
    return _gru(x, parts, W_ih, W_hh, b_ih, b_hh)
